# baseline (speedup 1.0000x reference)
import functools

import jax
import jax.numpy as jnp
from jax import lax
from jax.experimental import pallas as pl
from jax.experimental.pallas import tpu as pltpu
from jax.experimental.pallas import tpu_sc as plsc

N = 320000
D = 128
S = 10000
NC = 2
NS = 16
CHUNK = 128
NCHUNK = N // CHUNK
SUBCHUNKS = 1
ZROWS = 80
NZBLK = S // ZROWS


def _sc_partials(data, idx2d):
    mesh = plsc.VectorSubcoreMesh(core_axis_name="c", subcore_axis_name="s")

    @functools.partial(
        pl.kernel,
        out_type=jax.ShapeDtypeStruct((NC, S, D), jnp.float32),
        mesh=mesh,
        scratch_types=[
            pltpu.VMEM_SHARED((S, D), jnp.float32),
            pltpu.VMEM((ZROWS, D), jnp.float32),
        ],
    )
    def k(data_hbm, idx_hbm, out_hbm, acc, zbuf):
        cid = lax.axis_index("c")
        sid = lax.axis_index("s")

        @pl.loop(0, ZROWS)
        def _(r):
            @pl.loop(0, D // 16)
            def _(col):
                zbuf[r, pl.ds(col * 16, 16)] = jnp.zeros((16,), jnp.float32)

        @pl.loop(sid, NZBLK, step=NS)
        def _(b):
            pltpu.sync_copy(zbuf, acc.at[pl.ds(b * ZROWS, ZROWS)])

        plsc.subcore_barrier()

        def body(d_vmem, i_vmem):
            for u in range(SUBCHUNKS):
                if SUBCHUNKS == 1:
                    pltpu.sync_copy(d_vmem, acc.at[i_vmem.at[u]], add=True)
                else:
                    pltpu.sync_copy(d_vmem.at[pl.ds(u * CHUNK, CHUNK)],
                                    acc.at[i_vmem.at[u]], add=True)

        off = NTC // (SUBCHUNKS * CHUNK)
        pltpu.emit_pipeline(
            body,
            grid=((NCHUNK - NTC // CHUNK) // SUBCHUNKS,),
            in_specs=[
                pl.BlockSpec((SUBCHUNKS * CHUNK, D), lambda i: (i + off, 0)),
                pl.BlockSpec((SUBCHUNKS, CHUNK), lambda i: (i + off, 0)),
            ],
            out_specs=[],
            core_axis_name=("c", "s"),
            dimension_semantics=(pltpu.PARALLEL,),
        )(data_hbm, idx_hbm)

        plsc.subcore_barrier()

        @pl.loop(sid, NZBLK, step=NS)
        def _(b):
            pltpu.sync_copy(acc.at[pl.ds(b * ZROWS, ZROWS)],
                            out_hbm.at[cid, pl.ds(b * ZROWS, ZROWS)])

    return k(data, idx2d)


RTC = 8000
STC = 2000
WTC = 128
S_PAD = 10176
M_TC = 24
NTC = M_TC * RTC


def _tc_partial(data, ids3d, nb=None):
    if nb is None:
        nb = M_TC

    def body(ids_ref, x_ref, o_ref, acc_ref):
        b = pl.program_id(0)

        @pl.when(b == 0)
        def _():
            acc_ref[...] = jnp.zeros_like(acc_ref)

        ids = ids_ref[0]
        x = x_ref[...]

        for k in range(RTC // STC):
            idsk = ids[:, k * STC:(k + 1) * STC]
            xk = x[k * STC:(k + 1) * STC, :]
            last = idsk[0, STC - 1]
            idsb = jnp.broadcast_to(idsk, (WTC, STC))

            def cond(wb):
                return wb <= last

            def wbody(wb, idsk=idsk, idsb=idsb, xk=xk):
                wb = pl.multiple_of(wb, 8)
                col = jax.lax.broadcasted_iota(jnp.int32, (WTC, STC), 0) + wb
                oh = (col == idsb).astype(jnp.float32)
                p = jnp.dot(oh, xk, preferred_element_type=jnp.float32)
                acc_ref[pl.ds(wb, WTC), :] += p
                nxt = jnp.min(
                    jnp.where(idsk >= wb + WTC, idsk, jnp.int32(2**30)))
                return (nxt // 8) * 8

            jax.lax.while_loop(cond, wbody, (idsk[0, 0] // 8) * 8)

        @pl.when(b == nb - 1)
        def _():
            o_ref[...] = acc_ref[:S, :]

    return pl.pallas_call(
        body,
        grid=(nb,),
        in_specs=[
            pl.BlockSpec((1, 1, RTC), lambda i: (i, 0, 0)),
            pl.BlockSpec((RTC, D), lambda i: (i, 0)),
        ],
        out_specs=pl.BlockSpec((S, D), lambda i: (0, 0)),
        out_shape=jax.ShapeDtypeStruct((S, D), jnp.float32),
        scratch_shapes=[pltpu.VMEM((S_PAD, D), jnp.float32)],
        compiler_params=pltpu.CompilerParams(
            dimension_semantics=("arbitrary",)),
    )(ids3d, data)


def _combine(sc_partials, tc_partial):
    blk = 2000

    def add_body(a_ref, t_ref, o_ref):
        o_ref[...] = a_ref[0] + a_ref[1] + t_ref[...]

    return pl.pallas_call(
        add_body,
        out_shape=jax.ShapeDtypeStruct((S, D), jnp.float32),
        grid=(S // blk,),
        in_specs=[pl.BlockSpec((NC, blk, D), lambda i: (0, i, 0)),
                  pl.BlockSpec((blk, D), lambda i: (i, 0))],
        out_specs=pl.BlockSpec((blk, D), lambda i: (i, 0)),
    )(sc_partials, tc_partial)


def kernel(per_atom_pred, nodes_to_mol_index):
    ids3d = nodes_to_mol_index.reshape(N // RTC, 1, RTC)
    idx2d = nodes_to_mol_index.reshape(NCHUNK, CHUNK)
    sc_partials = _sc_partials(per_atom_pred, idx2d)
    tc_partial = _tc_partial(per_atom_pred, ids3d)
    return _combine(sc_partials, tc_partial)


# speedup vs baseline: 1.0217x; 1.0217x over previous
import functools

import jax
import jax.numpy as jnp
from jax import lax
from jax.experimental import pallas as pl
from jax.experimental.pallas import tpu as pltpu
from jax.experimental.pallas import tpu_sc as plsc

N = 320000
D = 128
S = 10000
NC = 2
NS = 16
CHUNK = 128
NCHUNK = N // CHUNK
SUBCHUNKS = 1
ZROWS = 80
NZBLK = S // ZROWS


def _sc_partials(data, idx2d):
    mesh = plsc.VectorSubcoreMesh(core_axis_name="c", subcore_axis_name="s")

    @functools.partial(
        pl.kernel,
        out_type=jax.ShapeDtypeStruct((NC, S, D), jnp.float32),
        mesh=mesh,
        scratch_types=[
            pltpu.VMEM_SHARED((S, D), jnp.float32),
            pltpu.VMEM((ZROWS, D), jnp.float32),
        ],
    )
    def k(data_hbm, idx_hbm, out_hbm, acc, zbuf):
        cid = lax.axis_index("c")
        sid = lax.axis_index("s")

        @pl.loop(0, ZROWS)
        def _(r):
            @pl.loop(0, D // 16)
            def _(col):
                zbuf[r, pl.ds(col * 16, 16)] = jnp.zeros((16,), jnp.float32)

        @pl.loop(sid, NZBLK, step=NS)
        def _(b):
            pltpu.sync_copy(zbuf, acc.at[pl.ds(b * ZROWS, ZROWS)])

        plsc.subcore_barrier()

        def body(d_vmem, i_vmem):
            for u in range(SUBCHUNKS):
                if SUBCHUNKS == 1:
                    pltpu.sync_copy(d_vmem, acc.at[i_vmem.at[u]], add=True)
                else:
                    pltpu.sync_copy(d_vmem.at[pl.ds(u * CHUNK, CHUNK)],
                                    acc.at[i_vmem.at[u]], add=True)

        off = NTC // (SUBCHUNKS * CHUNK)
        pltpu.emit_pipeline(
            body,
            grid=((NCHUNK - NTC // CHUNK) // SUBCHUNKS,),
            in_specs=[
                pl.BlockSpec((SUBCHUNKS * CHUNK, D), lambda i: (i + off, 0)),
                pl.BlockSpec((SUBCHUNKS, CHUNK), lambda i: (i + off, 0)),
            ],
            out_specs=[],
            core_axis_name=("c", "s"),
            dimension_semantics=(pltpu.PARALLEL,),
        )(data_hbm, idx_hbm)

        plsc.subcore_barrier()

        @pl.loop(sid, NZBLK, step=NS)
        def _(b):
            pltpu.sync_copy(acc.at[pl.ds(b * ZROWS, ZROWS)],
                            out_hbm.at[cid, pl.ds(b * ZROWS, ZROWS)])

    return k(data, idx2d)


RTC = 8192
STC = 2048
WTC = 128
S_PAD = 10176
M_TC = 24
NTC = M_TC * RTC


def _tc_partial(data, ids3d, nb=None):
    if nb is None:
        nb = M_TC

    def body(ids_ref, x_ref, o_ref, acc_ref):
        b = pl.program_id(0)

        @pl.when(b == 0)
        def _():
            acc_ref[...] = jnp.zeros_like(acc_ref)

        ids = ids_ref[0]
        x = x_ref[...]

        for k in range(RTC // STC):
            idsk = ids[:, k * STC:(k + 1) * STC]
            xk = x[k * STC:(k + 1) * STC, :]
            last = idsk[0, STC - 1]
            idsb = jnp.broadcast_to(idsk, (WTC, STC))

            def cond(wb):
                return wb <= last

            def wbody(wb, idsk=idsk, idsb=idsb, xk=xk):
                wb = pl.multiple_of(wb, 8)
                col = jax.lax.broadcasted_iota(jnp.int32, (WTC, STC), 0) + wb
                oh = (col == idsb).astype(jnp.float32)
                p = jnp.dot(oh, xk, preferred_element_type=jnp.float32)
                acc_ref[pl.ds(wb, WTC), :] += p
                nxt = jnp.min(
                    jnp.where(idsk >= wb + WTC, idsk, jnp.int32(2**30)))
                return (nxt // 8) * 8

            jax.lax.while_loop(cond, wbody, (idsk[0, 0] // 8) * 8)

        @pl.when(b == nb - 1)
        def _():
            o_ref[...] = acc_ref[:S, :]

    return pl.pallas_call(
        body,
        grid=(nb,),
        in_specs=[
            pl.BlockSpec((1, 1, RTC), lambda i: (i, 0, 0)),
            pl.BlockSpec((RTC, D), lambda i: (i, 0)),
        ],
        out_specs=pl.BlockSpec((S, D), lambda i: (0, 0)),
        out_shape=jax.ShapeDtypeStruct((S, D), jnp.float32),
        scratch_shapes=[pltpu.VMEM((S_PAD, D), jnp.float32)],
        compiler_params=pltpu.CompilerParams(
            dimension_semantics=("arbitrary",)),
    )(ids3d, data)


def _combine(sc_partials, tc_partial):
    blk = 1000

    def add_body(a_ref, t_ref, o_ref):
        o_ref[...] = a_ref[0] + a_ref[1] + t_ref[...]

    return pl.pallas_call(
        add_body,
        out_shape=jax.ShapeDtypeStruct((S, D), jnp.float32),
        grid=(S // blk,),
        in_specs=[pl.BlockSpec((NC, blk, D), lambda i: (0, i, 0)),
                  pl.BlockSpec((blk, D), lambda i: (i, 0))],
        out_specs=pl.BlockSpec((blk, D), lambda i: (i, 0)),
    )(sc_partials, tc_partial)


def kernel(per_atom_pred, nodes_to_mol_index):
    ids3d = nodes_to_mol_index[:NTC].reshape(M_TC, 1, RTC)
    idx2d = nodes_to_mol_index.reshape(NCHUNK, CHUNK)
    sc_partials = _sc_partials(per_atom_pred, idx2d)
    tc_partial = _tc_partial(per_atom_pred, ids3d)
    return _combine(sc_partials, tc_partial)
